# baseline (speedup 1.0000x reference)
import functools

import jax
import jax.numpy as jnp
from jax import lax
from jax.experimental import pallas as pl
from jax.experimental.pallas import tpu as pltpu
from jax.experimental.pallas import tpu_sc as plsc

_N = 10000
_N_PAD = 10240
_ROWS_PER_SUB = _N_PAD // 16
_BLK = 128
_FC = 128
_M_BLK = 512
_GRID_M = _N_PAD // _M_BLK

_MESH = plsc.VectorSubcoreMesh(core_axis_name="c", subcore_axis_name="s")
_HI = lax.Precision.HIGHEST


def _sc_degree(dst32, ones_blk):
    nblk = dst32.shape[1]

    @functools.partial(
        pl.kernel,
        out_type=jax.ShapeDtypeStruct((2, _N_PAD, 16), jnp.float32),
        mesh=_MESH,
        scratch_types=[
            pltpu.VMEM((nblk, _BLK), jnp.int32),
            pltpu.VMEM((_BLK, 16), jnp.float32),
            pltpu.VMEM_SHARED((_N_PAD, 16), jnp.float32),
        ],
    )
    def deg_kernel(dst_hbm, ones_hbm, out_hbm, dst_v, ones_v, acc):
        c = lax.axis_index("c")
        s = lax.axis_index("s")
        w = c * 16 + s
        pltpu.sync_copy(dst_hbm.at[w], dst_v)
        pltpu.sync_copy(ones_hbm, ones_v)
        base = s * _ROWS_PER_SUB
        for i in range(_ROWS_PER_SUB // _BLK):
            pltpu.sync_copy(ones_v, acc.at[pl.ds(base + i * _BLK, _BLK)])
        plsc.subcore_barrier()

        @pl.loop(0, nblk)
        def _(j):
            pltpu.sync_copy(ones_v, acc.at[dst_v.at[j]], add=True)

        plsc.subcore_barrier()
        pltpu.sync_copy(acc.at[pl.ds(base, _ROWS_PER_SUB)],
                        out_hbm.at[c].at[pl.ds(base, _ROWS_PER_SUB)])

    return deg_kernel(dst32, ones_blk)


_G = 32
_L = 2560
_NQ = _L // _G
_NB = 10


def _edge_sweep(xs_ref, acc, src_v, dst_v, bufs, gsems, ssems):
    def gath(t, q):
        return pltpu.make_async_copy(
            xs_ref.at[src_v.at[pl.ds(q * _G, _G)]], bufs[t], gsems[t])

    def scat(t, q):
        return pltpu.make_async_copy(
            bufs[t], acc.at[dst_v.at[pl.ds(q * _G, _G)]], ssems[t])

    for t in range(_NB):
        gath(t, t).start()

    @pl.loop(0, _NQ, step=_NB)
    def _(q):
        for t in range(_NB):
            gath(t, q + t).wait()
            scat(t, q + t).start(add=True)
        for t in range(_NB):
            @pl.when(q + _NB + t < _NQ)
            def _(t=t):
                scat(t, q + t).wait()
                gath(t, q + _NB + t).start()

    for t in range(_NB):
        scat(t, _NQ - _NB + t).wait()


def _prop_scratch(fc):
    return (
        [pltpu.VMEM((_L,), jnp.int32), pltpu.VMEM((_L,), jnp.int32)]
        + [pltpu.VMEM((_G, fc), jnp.float32) for _ in range(_NB)]
        + [pltpu.VMEM_SHARED((_N_PAD, fc), jnp.float32)]
        + [pltpu.SemaphoreType.DMA for _ in range(2 * _NB)]
    )


def _sc_propagate(xs, src16, dst16):
    ncc, _, fc = xs.shape
    nsweep = src16.shape[1] // _L

    @functools.partial(
        pl.kernel,
        out_type=jax.ShapeDtypeStruct((ncc, _N_PAD, fc), jnp.float32),
        mesh=_MESH,
        scratch_types=_prop_scratch(fc),
    )
    def prop_kernel(xs_hbm, src_hbm, dst_hbm, out_hbm,
                    src_v, dst_v, *rest):
        bufs, acc, sems = rest[:_NB], rest[_NB], rest[_NB + 1:]
        gsems, ssems = sems[:_NB], sems[_NB:]
        c = lax.axis_index("c")
        s = lax.axis_index("s")
        base = s * _ROWS_PER_SUB

        for k2 in range(ncc // 2):
            chunk = k2 * 2 + c
            xs_c = xs_hbm.at[chunk]
            pltpu.sync_copy(xs_c.at[pl.ds(base, _ROWS_PER_SUB)],
                            acc.at[pl.ds(base, _ROWS_PER_SUB)])
            plsc.subcore_barrier()

            for h in range(nsweep):
                pltpu.sync_copy(src_hbm.at[s].at[pl.ds(h * _L, _L)], src_v)
                pltpu.sync_copy(dst_hbm.at[s].at[pl.ds(h * _L, _L)], dst_v)
                _edge_sweep(xs_c, acc, src_v, dst_v, bufs, gsems, ssems)

            plsc.subcore_barrier()
            pltpu.sync_copy(acc.at[pl.ds(base, _ROWS_PER_SUB)],
                            out_hbm.at[chunk].at[pl.ds(base, _ROWS_PER_SUB)])
            plsc.subcore_barrier()

    return prop_kernel(xs, src16, dst16)


def _sc_propagate_split(xs, src32, dst32):
    fc = xs.shape[1]
    nsweep = src32.shape[1] // _L

    @functools.partial(
        pl.kernel,
        out_type=jax.ShapeDtypeStruct((2, _N_PAD, fc), jnp.float32),
        mesh=_MESH,
        scratch_types=_prop_scratch(fc),
    )
    def prop_kernel(xs_hbm, src_hbm, dst_hbm, out_hbm,
                    src_v, dst_v, *rest):
        bufs, acc, sems = rest[:_NB], rest[_NB], rest[_NB + 1:]
        gsems, ssems = sems[:_NB], sems[_NB:]
        c = lax.axis_index("c")
        s = lax.axis_index("s")
        w = c * 16 + s
        base = s * _ROWS_PER_SUB
        pltpu.sync_copy(xs_hbm.at[pl.ds(base, _ROWS_PER_SUB)],
                        acc.at[pl.ds(base, _ROWS_PER_SUB)])
        plsc.subcore_barrier()

        for h in range(nsweep):
            pltpu.sync_copy(src_hbm.at[w].at[pl.ds(h * _L, _L)], src_v)
            pltpu.sync_copy(dst_hbm.at[w].at[pl.ds(h * _L, _L)], dst_v)
            _edge_sweep(xs_hbm, acc, src_v, dst_v, bufs, gsems, ssems)

        plsc.subcore_barrier()
        pltpu.sync_copy(acc.at[pl.ds(base, _ROWS_PER_SUB)],
                        out_hbm.at[c].at[pl.ds(base, _ROWS_PER_SUB)])

    return prop_kernel(xs, src32, dst32)


def _tc_prep_x(x_pad, deg2):
    nf = x_pad.shape[1]
    ncc = nf // _FC

    def body(x_ref, deg_ref, v_ref, out_ref):
        d = deg_ref[0, :, 0:1] + deg_ref[1, :, 0:1] - 1.0
        v = lax.rsqrt(d)
        v_ref[...] = v
        xs = x_ref[...] * v
        for k in range(ncc):
            out_ref[k] = xs[:, k * _FC:(k + 1) * _FC]

    return pl.pallas_call(
        body,
        grid=(_GRID_M,),
        in_specs=[pl.BlockSpec((_M_BLK, nf), lambda i: (i, 0)),
                  pl.BlockSpec((2, _M_BLK, 16), lambda i: (0, i, 0))],
        out_specs=[pl.BlockSpec((_M_BLK, 1), lambda i: (i, 0)),
                   pl.BlockSpec((ncc, _M_BLK, _FC), lambda i: (0, i, 0))],
        out_shape=[jax.ShapeDtypeStruct((_N_PAD, 1), jnp.float32),
                   jax.ShapeDtypeStruct((ncc, _N_PAD, _FC), jnp.float32)],
    )(x_pad, deg2)


def _tc_fused12(acc1, dinv, W1, b1, W2):
    nci, _, fci = acc1.shape
    nh = W2.shape[1]
    nco = nh // _FC

    def body(p_ref, v_ref, w1_ref, b1_ref, w2_ref, out_ref):
        a = jnp.concatenate([p_ref[k] for k in range(nci)], axis=1)
        a = a * v_ref[...]
        h1 = jnp.dot(a, w1_ref[...], precision=_HI,
                     preferred_element_type=jnp.float32) + b1_ref[...]
        h1 = jnp.maximum(h1, 0.0)
        t2 = jnp.dot(h1, w2_ref[...], precision=_HI,
                     preferred_element_type=jnp.float32) * v_ref[...]
        for k in range(nco):
            out_ref[k] = t2[:, k * _FC:(k + 1) * _FC]

    return pl.pallas_call(
        body,
        grid=(_GRID_M,),
        in_specs=[
            pl.BlockSpec((nci, _M_BLK, fci), lambda i: (0, i, 0)),
            pl.BlockSpec((_M_BLK, 1), lambda i: (i, 0)),
            pl.BlockSpec(W1.shape, lambda i: (0, 0)),
            pl.BlockSpec(b1.shape, lambda i: (0, 0)),
            pl.BlockSpec(W2.shape, lambda i: (0, 0)),
        ],
        out_specs=pl.BlockSpec((nco, _M_BLK, _FC), lambda i: (0, i, 0)),
        out_shape=jax.ShapeDtypeStruct((nco, _N_PAD, _FC), jnp.float32),
    )(acc1, dinv, W1, b1, W2)


def _tc_fused3(acc2, dinv, b2, W3):
    nci, _, fci = acc2.shape
    ncl = W3.shape[1]

    def body(p_ref, v_ref, b2_ref, w3_ref, out_ref):
        a = jnp.concatenate([p_ref[k] for k in range(nci)], axis=1)
        h2 = jnp.maximum(a * v_ref[...] + b2_ref[...], 0.0)
        out_ref[...] = jnp.dot(h2, w3_ref[...], precision=_HI,
                               preferred_element_type=jnp.float32) * v_ref[...]

    return pl.pallas_call(
        body,
        grid=(_GRID_M,),
        in_specs=[
            pl.BlockSpec((nci, _M_BLK, fci), lambda i: (0, i, 0)),
            pl.BlockSpec((_M_BLK, 1), lambda i: (i, 0)),
            pl.BlockSpec(b2.shape, lambda i: (0, 0)),
            pl.BlockSpec(W3.shape, lambda i: (0, 0)),
        ],
        out_specs=pl.BlockSpec((_M_BLK, ncl), lambda i: (i, 0)),
        out_shape=jax.ShapeDtypeStruct((_N_PAD, ncl), jnp.float32),
    )(acc2, dinv, b2, W3)


def _tc_final(p3, t3s, dinv, b3):
    fc = p3.shape[2]

    def body(p_ref, x_ref, v_ref, b3_ref, out_ref):
        t = p_ref[0] + p_ref[1] - x_ref[...]
        t = t * v_ref[...] + b3_ref[...]
        m = jnp.max(t, axis=1, keepdims=True)
        e = jnp.exp(t - m)
        ssum = jnp.sum(e, axis=1, keepdims=True)
        out_ref[...] = t - m - jnp.log(ssum)

    return pl.pallas_call(
        body,
        grid=(_GRID_M,),
        in_specs=[
            pl.BlockSpec((2, _M_BLK, fc), lambda i: (0, i, 0)),
            pl.BlockSpec((_M_BLK, fc), lambda i: (i, 0)),
            pl.BlockSpec((_M_BLK, 1), lambda i: (i, 0)),
            pl.BlockSpec(b3.shape, lambda i: (0, 0)),
        ],
        out_specs=pl.BlockSpec((_M_BLK, fc), lambda i: (i, 0)),
        out_shape=jax.ShapeDtypeStruct((_N_PAD, fc), jnp.float32),
    )(p3, t3s, dinv, b3)


def kernel(x, edge_index, W1, b1, W2, b2, W3, b3):
    e = edge_index.shape[1]
    unit = 32 * _L
    e_pad = ((e + unit - 1) // unit) * unit
    pad = jnp.full((e_pad - e,), _N, dtype=jnp.int32)
    src_p = jnp.concatenate([edge_index[0], pad])
    dst_p = jnp.concatenate([edge_index[1], pad])
    src16 = src_p.reshape(16, -1)
    dst16 = dst_p.reshape(16, -1)
    src32 = src_p.reshape(32, -1)
    dst32 = dst_p.reshape(32, -1)
    dst32b = dst_p.reshape(32, -1, _BLK)
    ones_blk = jnp.ones((_BLK, 16), jnp.float32)

    deg2 = _sc_degree(dst32b, ones_blk)

    x_pad = jnp.pad(x, ((0, _N_PAD - x.shape[0]), (0, 0)))
    dinv, xs = _tc_prep_x(x_pad, deg2)
    acc1 = _sc_propagate(xs, src16, dst16)
    t2s = _tc_fused12(acc1, dinv, W1, b1.reshape(1, -1), W2)
    acc2 = _sc_propagate(t2s, src16, dst16)
    t3s = _tc_fused3(acc2, dinv, b2.reshape(1, -1), W3)
    p3 = _sc_propagate_split(t3s, src32, dst32)
    out = _tc_final(p3, t3s, dinv, b3.reshape(1, -1))
    return out[:x.shape[0]]


# speedup vs baseline: 1.0057x; 1.0057x over previous
import functools

import jax
import jax.numpy as jnp
from jax import lax
from jax.experimental import pallas as pl
from jax.experimental.pallas import tpu as pltpu
from jax.experimental.pallas import tpu_sc as plsc

_N = 10000
_N_PAD = 10240
_ROWS_PER_SUB = _N_PAD // 16
_BLK = 128
_FC = 128
_M_BLK = 512
_GRID_M = _N_PAD // _M_BLK

_MESH = plsc.VectorSubcoreMesh(core_axis_name="c", subcore_axis_name="s")
_HI = lax.Precision.HIGHEST


def _sc_degree(dst32, ones_blk):
    nblk = dst32.shape[1]

    @functools.partial(
        pl.kernel,
        out_type=jax.ShapeDtypeStruct((2, _N_PAD, 16), jnp.float32),
        mesh=_MESH,
        scratch_types=[
            pltpu.VMEM((nblk, _BLK), jnp.int32),
            pltpu.VMEM((_BLK, 16), jnp.float32),
            pltpu.VMEM_SHARED((_N_PAD, 16), jnp.float32),
        ],
    )
    def deg_kernel(dst_hbm, ones_hbm, out_hbm, dst_v, ones_v, acc):
        c = lax.axis_index("c")
        s = lax.axis_index("s")
        w = c * 16 + s
        pltpu.sync_copy(dst_hbm.at[w], dst_v)
        pltpu.sync_copy(ones_hbm, ones_v)
        base = s * _ROWS_PER_SUB
        for i in range(_ROWS_PER_SUB // _BLK):
            pltpu.sync_copy(ones_v, acc.at[pl.ds(base + i * _BLK, _BLK)])
        plsc.subcore_barrier()

        @pl.loop(0, nblk)
        def _(j):
            pltpu.sync_copy(ones_v, acc.at[dst_v.at[j]], add=True)

        plsc.subcore_barrier()
        pltpu.sync_copy(acc.at[pl.ds(base, _ROWS_PER_SUB)],
                        out_hbm.at[c].at[pl.ds(base, _ROWS_PER_SUB)])

    return deg_kernel(dst32, ones_blk)


_G = 32
_L = 5120
_NQ = _L // _G
_NB = 8


def _edge_sweep(xs_ref, acc, src_v, dst_v, bufs, gsems, ssems):
    def gath(t, q):
        return pltpu.make_async_copy(
            xs_ref.at[src_v.at[pl.ds(q * _G, _G)]], bufs[t], gsems[t])

    def scat(t, q):
        return pltpu.make_async_copy(
            bufs[t], acc.at[dst_v.at[pl.ds(q * _G, _G)]], ssems[t])

    for t in range(_NB):
        gath(t, t).start()

    @pl.loop(0, _NQ, step=_NB)
    def _(q):
        for t in range(_NB):
            gath(t, q + t).wait()
            scat(t, q + t).start(add=True)
        for t in range(_NB):
            @pl.when(q + _NB + t < _NQ)
            def _(t=t):
                scat(t, q + t).wait()
                gath(t, q + _NB + t).start()

    for t in range(_NB):
        scat(t, _NQ - _NB + t).wait()


def _prop_scratch(fc):
    return (
        [pltpu.VMEM((_L,), jnp.int32), pltpu.VMEM((_L,), jnp.int32)]
        + [pltpu.VMEM((_G, fc), jnp.float32) for _ in range(_NB)]
        + [pltpu.VMEM_SHARED((_N_PAD, fc), jnp.float32)]
        + [pltpu.SemaphoreType.DMA for _ in range(2 * _NB)]
    )


def _sc_propagate(xs, src16, dst16):
    ncc, _, fc = xs.shape
    nsweep = src16.shape[1] // _L

    @functools.partial(
        pl.kernel,
        out_type=jax.ShapeDtypeStruct((ncc, _N_PAD, fc), jnp.float32),
        mesh=_MESH,
        scratch_types=_prop_scratch(fc),
    )
    def prop_kernel(xs_hbm, src_hbm, dst_hbm, out_hbm,
                    src_v, dst_v, *rest):
        bufs, acc, sems = rest[:_NB], rest[_NB], rest[_NB + 1:]
        gsems, ssems = sems[:_NB], sems[_NB:]
        c = lax.axis_index("c")
        s = lax.axis_index("s")
        base = s * _ROWS_PER_SUB

        for k2 in range(ncc // 2):
            chunk = k2 * 2 + c
            xs_c = xs_hbm.at[chunk]
            pltpu.sync_copy(xs_c.at[pl.ds(base, _ROWS_PER_SUB)],
                            acc.at[pl.ds(base, _ROWS_PER_SUB)])
            plsc.subcore_barrier()

            for h in range(nsweep):
                pltpu.sync_copy(src_hbm.at[s].at[pl.ds(h * _L, _L)], src_v)
                pltpu.sync_copy(dst_hbm.at[s].at[pl.ds(h * _L, _L)], dst_v)
                _edge_sweep(xs_c, acc, src_v, dst_v, bufs, gsems, ssems)

            plsc.subcore_barrier()
            pltpu.sync_copy(acc.at[pl.ds(base, _ROWS_PER_SUB)],
                            out_hbm.at[chunk].at[pl.ds(base, _ROWS_PER_SUB)])
            plsc.subcore_barrier()

    return prop_kernel(xs, src16, dst16)


def _sc_propagate_split(xs, src32, dst32):
    fc = xs.shape[1]
    nsweep = src32.shape[1] // _L

    @functools.partial(
        pl.kernel,
        out_type=jax.ShapeDtypeStruct((2, _N_PAD, fc), jnp.float32),
        mesh=_MESH,
        scratch_types=_prop_scratch(fc),
    )
    def prop_kernel(xs_hbm, src_hbm, dst_hbm, out_hbm,
                    src_v, dst_v, *rest):
        bufs, acc, sems = rest[:_NB], rest[_NB], rest[_NB + 1:]
        gsems, ssems = sems[:_NB], sems[_NB:]
        c = lax.axis_index("c")
        s = lax.axis_index("s")
        w = c * 16 + s
        base = s * _ROWS_PER_SUB
        pltpu.sync_copy(xs_hbm.at[pl.ds(base, _ROWS_PER_SUB)],
                        acc.at[pl.ds(base, _ROWS_PER_SUB)])
        plsc.subcore_barrier()

        for h in range(nsweep):
            pltpu.sync_copy(src_hbm.at[w].at[pl.ds(h * _L, _L)], src_v)
            pltpu.sync_copy(dst_hbm.at[w].at[pl.ds(h * _L, _L)], dst_v)
            _edge_sweep(xs_hbm, acc, src_v, dst_v, bufs, gsems, ssems)

        plsc.subcore_barrier()
        pltpu.sync_copy(acc.at[pl.ds(base, _ROWS_PER_SUB)],
                        out_hbm.at[c].at[pl.ds(base, _ROWS_PER_SUB)])

    return prop_kernel(xs, src32, dst32)


def _tc_prep_x(x_pad, deg2):
    nf = x_pad.shape[1]
    ncc = nf // _FC

    def body(x_ref, deg_ref, v_ref, out_ref):
        d = deg_ref[0, :, 0:1] + deg_ref[1, :, 0:1] - 1.0
        v = lax.rsqrt(d)
        v_ref[...] = v
        xs = x_ref[...] * v
        for k in range(ncc):
            out_ref[k] = xs[:, k * _FC:(k + 1) * _FC]

    return pl.pallas_call(
        body,
        grid=(_GRID_M,),
        in_specs=[pl.BlockSpec((_M_BLK, nf), lambda i: (i, 0)),
                  pl.BlockSpec((2, _M_BLK, 16), lambda i: (0, i, 0))],
        out_specs=[pl.BlockSpec((_M_BLK, 1), lambda i: (i, 0)),
                   pl.BlockSpec((ncc, _M_BLK, _FC), lambda i: (0, i, 0))],
        out_shape=[jax.ShapeDtypeStruct((_N_PAD, 1), jnp.float32),
                   jax.ShapeDtypeStruct((ncc, _N_PAD, _FC), jnp.float32)],
    )(x_pad, deg2)


def _tc_fused12(acc1, dinv, W1, b1, W2):
    nci, _, fci = acc1.shape
    nh = W2.shape[1]
    nco = nh // _FC

    def body(p_ref, v_ref, w1_ref, b1_ref, w2_ref, out_ref):
        a = jnp.concatenate([p_ref[k] for k in range(nci)], axis=1)
        a = a * v_ref[...]
        h1 = jnp.dot(a, w1_ref[...], precision=_HI,
                     preferred_element_type=jnp.float32) + b1_ref[...]
        h1 = jnp.maximum(h1, 0.0)
        t2 = jnp.dot(h1, w2_ref[...], precision=_HI,
                     preferred_element_type=jnp.float32) * v_ref[...]
        for k in range(nco):
            out_ref[k] = t2[:, k * _FC:(k + 1) * _FC]

    return pl.pallas_call(
        body,
        grid=(_GRID_M,),
        in_specs=[
            pl.BlockSpec((nci, _M_BLK, fci), lambda i: (0, i, 0)),
            pl.BlockSpec((_M_BLK, 1), lambda i: (i, 0)),
            pl.BlockSpec(W1.shape, lambda i: (0, 0)),
            pl.BlockSpec(b1.shape, lambda i: (0, 0)),
            pl.BlockSpec(W2.shape, lambda i: (0, 0)),
        ],
        out_specs=pl.BlockSpec((nco, _M_BLK, _FC), lambda i: (0, i, 0)),
        out_shape=jax.ShapeDtypeStruct((nco, _N_PAD, _FC), jnp.float32),
    )(acc1, dinv, W1, b1, W2)


def _tc_fused3(acc2, dinv, b2, W3):
    nci, _, fci = acc2.shape
    ncl = W3.shape[1]

    def body(p_ref, v_ref, b2_ref, w3_ref, out_ref):
        a = jnp.concatenate([p_ref[k] for k in range(nci)], axis=1)
        h2 = jnp.maximum(a * v_ref[...] + b2_ref[...], 0.0)
        out_ref[...] = jnp.dot(h2, w3_ref[...], precision=_HI,
                               preferred_element_type=jnp.float32) * v_ref[...]

    return pl.pallas_call(
        body,
        grid=(_GRID_M,),
        in_specs=[
            pl.BlockSpec((nci, _M_BLK, fci), lambda i: (0, i, 0)),
            pl.BlockSpec((_M_BLK, 1), lambda i: (i, 0)),
            pl.BlockSpec(b2.shape, lambda i: (0, 0)),
            pl.BlockSpec(W3.shape, lambda i: (0, 0)),
        ],
        out_specs=pl.BlockSpec((_M_BLK, ncl), lambda i: (i, 0)),
        out_shape=jax.ShapeDtypeStruct((_N_PAD, ncl), jnp.float32),
    )(acc2, dinv, b2, W3)


def _tc_final(p3, t3s, dinv, b3):
    fc = p3.shape[2]

    def body(p_ref, x_ref, v_ref, b3_ref, out_ref):
        t = p_ref[0] + p_ref[1] - x_ref[...]
        t = t * v_ref[...] + b3_ref[...]
        m = jnp.max(t, axis=1, keepdims=True)
        e = jnp.exp(t - m)
        ssum = jnp.sum(e, axis=1, keepdims=True)
        out_ref[...] = t - m - jnp.log(ssum)

    return pl.pallas_call(
        body,
        grid=(_GRID_M,),
        in_specs=[
            pl.BlockSpec((2, _M_BLK, fc), lambda i: (0, i, 0)),
            pl.BlockSpec((_M_BLK, fc), lambda i: (i, 0)),
            pl.BlockSpec((_M_BLK, 1), lambda i: (i, 0)),
            pl.BlockSpec(b3.shape, lambda i: (0, 0)),
        ],
        out_specs=pl.BlockSpec((_M_BLK, fc), lambda i: (i, 0)),
        out_shape=jax.ShapeDtypeStruct((_N_PAD, fc), jnp.float32),
    )(p3, t3s, dinv, b3)


def kernel(x, edge_index, W1, b1, W2, b2, W3, b3):
    e = edge_index.shape[1]
    unit = 32 * _L
    e_pad = ((e + unit - 1) // unit) * unit
    pad = jnp.full((e_pad - e,), _N, dtype=jnp.int32)
    src_p = jnp.concatenate([edge_index[0], pad])
    dst_p = jnp.concatenate([edge_index[1], pad])
    src16 = src_p.reshape(16, -1)
    dst16 = dst_p.reshape(16, -1)
    src32 = src_p.reshape(32, -1)
    dst32 = dst_p.reshape(32, -1)
    dst32b = dst_p.reshape(32, -1, _BLK)
    ones_blk = jnp.ones((_BLK, 16), jnp.float32)

    deg2 = _sc_degree(dst32b, ones_blk)

    x_pad = jnp.pad(x, ((0, _N_PAD - x.shape[0]), (0, 0)))
    dinv, xs = _tc_prep_x(x_pad, deg2)
    acc1 = _sc_propagate(xs, src16, dst16)
    t2s = _tc_fused12(acc1, dinv, W1, b1.reshape(1, -1), W2)
    acc2 = _sc_propagate(t2s, src16, dst16)
    t3s = _tc_fused3(acc2, dinv, b2.reshape(1, -1), W3)
    p3 = _sc_propagate_split(t3s, src32, dst32)
    out = _tc_final(p3, t3s, dinv, b3.reshape(1, -1))
    return out[:x.shape[0]]
